# baseline (speedup 1.0000x reference)
import functools

import jax
import jax.numpy as jnp
from jax import lax
from jax.experimental import pallas as pl
from jax.experimental.pallas import tpu as pltpu
from jax.experimental.pallas import tpu_sc as plsc

SEQ = 12
IMG = 32
EMB_D = IMG * IMG
X_CH = 3
X_COLS = X_CH * SEQ * EMB_D
O_COLS = (X_CH + 1) * SEQ * EMB_D


def _sc_gather(table, ids):
    n_rows, d = table.shape
    b = ids.shape[0]
    info = plsc.get_sparse_core_info()
    nw = info.num_cores * info.num_subcores
    b_per_w = b // nw

    mesh = plsc.VectorSubcoreMesh(core_axis_name="c", subcore_axis_name="s")

    @functools.partial(
        pl.kernel,
        mesh=mesh,
        out_type=jax.ShapeDtypeStruct((b, d), jnp.float32),
        scratch_types=[
            pltpu.VMEM((b_per_w,), jnp.int32),
            pltpu.VMEM((b_per_w, d), jnp.float32),
            pltpu.SemaphoreType.DMA,
        ],
    )
    def gather_kernel(table_hbm, idx_hbm, out_hbm, idx_v, rows_v, sem):
        wid = lax.axis_index("s") * info.num_cores + lax.axis_index("c")
        base = wid * b_per_w
        pltpu.sync_copy(idx_hbm.at[pl.ds(base, b_per_w)], idx_v)
        pltpu.async_copy(table_hbm.at[idx_v], rows_v, sem).wait()
        pltpu.sync_copy(rows_v, out_hbm.at[pl.ds(base, b_per_w)])

    return gather_kernel(table, ids)


def _assemble(x2, emb):
    b = x2.shape[0]
    bb = 16

    def body(x_ref, e_ref, o_ref):
        o_ref[:, :X_COLS] = x_ref[...]
        e = e_ref[...]
        for s in range(SEQ):
            o_ref[:, X_COLS + s * EMB_D:X_COLS + (s + 1) * EMB_D] = e

    return pl.pallas_call(
        body,
        grid=(b // bb,),
        in_specs=[
            pl.BlockSpec((bb, X_COLS), lambda i: (i, 0)),
            pl.BlockSpec((bb, EMB_D), lambda i: (i, 0)),
        ],
        out_specs=pl.BlockSpec((bb, O_COLS), lambda i: (i, 0)),
        out_shape=jax.ShapeDtypeStruct((b, O_COLS), jnp.float32),
    )(x2, emb)


def kernel(x, id, table):
    b = x.shape[0]
    emb = _sc_gather(table, id)
    x2 = x.reshape(b, X_COLS)
    out2 = _assemble(x2, emb)
    return out2.reshape(b, X_CH + 1, SEQ, IMG, IMG)


# speedup vs baseline: 3.1274x; 3.1274x over previous
import functools

import jax
import jax.numpy as jnp
from jax import lax
from jax.experimental import pallas as pl
from jax.experimental.pallas import tpu as pltpu
from jax.experimental.pallas import tpu_sc as plsc

SEQ = 12
IMG = 32
EMB_D = IMG * IMG
X_CH = 3
X_COLS = X_CH * SEQ * EMB_D
O_COLS = (X_CH + 1) * SEQ * EMB_D


def _sc_gather(table, ids):
    n_rows, d = table.shape
    b = ids.shape[0]
    info = plsc.get_sparse_core_info()
    nw = info.num_cores * info.num_subcores
    b_per_w = b // nw

    mesh = plsc.VectorSubcoreMesh(core_axis_name="c", subcore_axis_name="s")

    @functools.partial(
        pl.kernel,
        mesh=mesh,
        out_type=jax.ShapeDtypeStruct((b, d), jnp.float32),
        scratch_types=[
            pltpu.VMEM((b_per_w,), jnp.int32),
            pltpu.VMEM((b_per_w, d), jnp.float32),
            pltpu.SemaphoreType.DMA,
        ],
    )
    def gather_kernel(table_hbm, idx_hbm, out_hbm, idx_v, rows_v, sem):
        wid = lax.axis_index("s") * info.num_cores + lax.axis_index("c")
        base = wid * b_per_w
        pltpu.sync_copy(idx_hbm.at[pl.ds(base, b_per_w)], idx_v)
        pltpu.async_copy(table_hbm.at[idx_v], rows_v, sem).wait()
        pltpu.sync_copy(rows_v, out_hbm.at[pl.ds(base, b_per_w)])

    return gather_kernel(table, ids)


def _assemble_t(xt, embt):
    b = xt.shape[1]
    n_xblk = X_COLS // EMB_D
    n_blk = O_COLS // EMB_D

    def body(x_ref, e_ref, o_ref):
        i = pl.program_id(0)

        @pl.when(i < n_xblk)
        def _():
            o_ref[...] = x_ref[...]

        @pl.when(i >= n_xblk)
        def _():
            o_ref[...] = e_ref[...]

    return pl.pallas_call(
        body,
        grid=(n_blk,),
        in_specs=[
            pl.BlockSpec((EMB_D, b), lambda i: (jnp.minimum(i, n_xblk - 1), 0)),
            pl.BlockSpec((EMB_D, b), lambda i: (0, 0)),
        ],
        out_specs=pl.BlockSpec((EMB_D, b), lambda i: (i, 0)),
        out_shape=jax.ShapeDtypeStruct((O_COLS, b), jnp.float32),
    )(xt, embt)


def kernel(x, id, table):
    b = x.shape[0]
    emb = _sc_gather(table, id)
    xt = x.reshape(b, X_COLS).T
    embt = emb.T
    outt = _assemble_t(xt, embt)
    return outt.T.reshape(b, X_CH + 1, SEQ, IMG, IMG)


# speedup vs baseline: 3.1903x; 1.0201x over previous
import functools

import jax
import jax.numpy as jnp
from jax import lax
from jax.experimental import pallas as pl
from jax.experimental.pallas import tpu as pltpu
from jax.experimental.pallas import tpu_sc as plsc

SEQ = 12
IMG = 32
EMB_D = IMG * IMG
X_CH = 3
X_COLS = X_CH * SEQ * EMB_D
O_COLS = (X_CH + 1) * SEQ * EMB_D


def _sc_gather(table, ids):
    n_rows, d = table.shape
    b = ids.shape[0]
    info = plsc.get_sparse_core_info()
    nw = info.num_cores * info.num_subcores
    b_per_w = b // nw

    mesh = plsc.VectorSubcoreMesh(core_axis_name="c", subcore_axis_name="s")

    @functools.partial(
        pl.kernel,
        mesh=mesh,
        out_type=jax.ShapeDtypeStruct((b, d), jnp.float32),
        scratch_types=[
            pltpu.VMEM((b_per_w,), jnp.int32),
            pltpu.VMEM((b_per_w, d), jnp.float32),
            pltpu.SemaphoreType.DMA,
        ],
    )
    def gather_kernel(table_hbm, idx_hbm, out_hbm, idx_v, rows_v, sem):
        wid = lax.axis_index("s") * info.num_cores + lax.axis_index("c")
        base = wid * b_per_w
        pltpu.sync_copy(idx_hbm.at[pl.ds(base, b_per_w)], idx_v)
        pltpu.async_copy(table_hbm.at[idx_v], rows_v, sem).wait()
        pltpu.sync_copy(rows_v, out_hbm.at[pl.ds(base, b_per_w)])

    return gather_kernel(table, ids)


def _copy_x(xt):
    b = xt.shape[1]
    n_xblk = X_COLS // EMB_D

    def body(x_ref, o_ref):
        o_ref[...] = x_ref[...]

    return pl.pallas_call(
        body,
        grid=(n_xblk,),
        in_specs=[pl.BlockSpec((EMB_D, b), lambda i: (i, 0))],
        out_specs=pl.BlockSpec((EMB_D, b), lambda i: (i, 0)),
        out_shape=jax.ShapeDtypeStruct((O_COLS, b), jnp.float32),
    )(xt)


def _fill_emb(buf, embt):
    b = embt.shape[1]
    n_xblk = X_COLS // EMB_D

    def body(buf_ref, e_ref, o_ref):
        o_ref[...] = e_ref[...]

    return pl.pallas_call(
        body,
        grid=(SEQ,),
        in_specs=[
            pl.BlockSpec(memory_space=pl.ANY),
            pl.BlockSpec((EMB_D, b), lambda j: (0, 0)),
        ],
        out_specs=pl.BlockSpec((EMB_D, b), lambda j: (n_xblk + j, 0)),
        out_shape=jax.ShapeDtypeStruct((O_COLS, b), jnp.float32),
        input_output_aliases={0: 0},
    )(buf, embt)


def kernel(x, id, table):
    b = x.shape[0]
    emb = _sc_gather(table, id)
    xt = x.reshape(b, X_COLS).T
    embt = emb.T
    buf = _copy_x(xt)
    outt = _fill_emb(buf, embt)
    return outt.T.reshape(b, X_CH + 1, SEQ, IMG, IMG)


# speedup vs baseline: 3.2108x; 1.0064x over previous
import functools

import jax
import jax.numpy as jnp
from jax import lax
from jax.experimental import pallas as pl
from jax.experimental.pallas import tpu as pltpu
from jax.experimental.pallas import tpu_sc as plsc

SEQ = 12
IMG = 32
EMB_D = IMG * IMG
X_CH = 3
X_COLS = X_CH * SEQ * EMB_D
O_COLS = (X_CH + 1) * SEQ * EMB_D


def _sc_gather(table, ids):
    n_rows, d = table.shape
    b = ids.shape[0]
    info = plsc.get_sparse_core_info()
    nw = info.num_cores * info.num_subcores
    b_per_w = b // nw

    mesh = plsc.VectorSubcoreMesh(core_axis_name="c", subcore_axis_name="s")

    @functools.partial(
        pl.kernel,
        mesh=mesh,
        out_type=jax.ShapeDtypeStruct((b, d), jnp.float32),
        scratch_types=[
            pltpu.VMEM((b_per_w,), jnp.int32),
            pltpu.VMEM((b_per_w, d), jnp.float32),
            pltpu.SemaphoreType.DMA,
        ],
    )
    def gather_kernel(table_hbm, idx_hbm, out_hbm, idx_v, rows_v, sem):
        wid = lax.axis_index("s") * info.num_cores + lax.axis_index("c")
        base = wid * b_per_w
        pltpu.sync_copy(idx_hbm.at[pl.ds(base, b_per_w)], idx_v)
        pltpu.async_copy(table_hbm.at[idx_v], rows_v, sem).wait()
        pltpu.sync_copy(rows_v, out_hbm.at[pl.ds(base, b_per_w)])

    return gather_kernel(table, ids)


def _copy_x(xt):
    b = xt.shape[1]
    rows = 2048
    n_xblk = X_COLS // rows

    def body(x_ref, o_ref):
        o_ref[...] = x_ref[...]

    return pl.pallas_call(
        body,
        grid=(n_xblk,),
        in_specs=[pl.BlockSpec((rows, b), lambda i: (i, 0))],
        out_specs=pl.BlockSpec((rows, b), lambda i: (i, 0)),
        out_shape=jax.ShapeDtypeStruct((O_COLS, b), jnp.float32),
    )(xt)


def _fill_emb(buf, embt):
    b = embt.shape[1]
    reps = 2
    rows = reps * EMB_D
    n_xblk = X_COLS // rows

    def body(buf_ref, e_ref, o_ref):
        e = e_ref[...]
        for k in range(reps):
            o_ref[k * EMB_D:(k + 1) * EMB_D, :] = e

    return pl.pallas_call(
        body,
        grid=(SEQ // reps,),
        in_specs=[
            pl.BlockSpec(memory_space=pl.ANY),
            pl.BlockSpec((EMB_D, b), lambda j: (0, 0)),
        ],
        out_specs=pl.BlockSpec((rows, b), lambda j: (n_xblk + j, 0)),
        out_shape=jax.ShapeDtypeStruct((O_COLS, b), jnp.float32),
        input_output_aliases={0: 0},
    )(buf, embt)


def kernel(x, id, table):
    b = x.shape[0]
    emb = _sc_gather(table, id)
    xt = x.reshape(b, X_COLS).T
    embt = emb.T
    buf = _copy_x(xt)
    outt = _fill_emb(buf, embt)
    return outt.T.reshape(b, X_CH + 1, SEQ, IMG, IMG)
